# Initial kernel scaffold; baseline (speedup 1.0000x reference)
import jax
import jax.numpy as jnp
from jax.experimental import pallas as pl


def kernel(x, table):
    raise NotImplementedError("write your pallas kernel here")


# speedup vs baseline: 1.4587x; 1.4587x over previous
import functools

import jax
import jax.numpy as jnp
from jax import lax
from jax.experimental import pallas as pl
from jax.experimental.pallas import tpu as pltpu
from jax.experimental.pallas import tpu_sc as plsc

_B, _L, _D = 4096, 200, 32
_N = _B * _L
_IW = 128
_KD = 8
_CHUNK = _IW * _KD


def _build():
    info = plsc.get_sparse_core_info()
    nc = info.num_cores
    nw = nc * info.num_subcores
    n_per_w = _N // nw
    nblk = n_per_w // _CHUNK
    rows_per_w = n_per_w // _IW
    mesh = plsc.VectorSubcoreMesh(core_axis_name="c", subcore_axis_name="s")

    @functools.partial(
        pl.kernel,
        mesh=mesh,
        out_type=jax.ShapeDtypeStruct((_N, _D), jnp.float32),
        compiler_params=pltpu.CompilerParams(use_tc_tiling_on_sc=False),
        scratch_types=[
            pltpu.VMEM((_KD, _IW), jnp.int32),
            pltpu.VMEM((_CHUNK, _D), jnp.float32),
            pltpu.SemaphoreType.DMA,
        ],
    )
    def gather(idx_hbm, table_hbm, out_hbm, idx_v, rows_v, sem):
        wid = lax.axis_index("s") * nc + lax.axis_index("c")
        row0 = wid * rows_per_w
        base0 = wid * n_per_w

        def body(i, carry):
            pltpu.sync_copy(idx_hbm.at[pl.ds(row0 + i * _KD, _KD)], idx_v)
            copies = [
                pltpu.async_copy(
                    table_hbm.at[idx_v.at[j]],
                    rows_v.at[pl.ds(j * _IW, _IW)],
                    sem,
                )
                for j in range(_KD)
            ]
            for c in copies:
                c.wait()
            pltpu.sync_copy(rows_v, out_hbm.at[pl.ds(base0 + i * _CHUNK, _CHUNK)])
            return carry

        lax.fori_loop(0, nblk, body, 0)

    return gather


_gather = _build()


def kernel(x, table):
    idx = x.reshape(_N // _IW, _IW)
    out = _gather(idx, table)
    return out.reshape(_B, _L, _D)


# speedup vs baseline: 1.5019x; 1.0296x over previous
import functools

import jax
import jax.numpy as jnp
from jax import lax
from jax.experimental import pallas as pl
from jax.experimental.pallas import tpu as pltpu
from jax.experimental.pallas import tpu_sc as plsc

_B, _L, _D = 4096, 200, 32
_N = _B * _L
_IW = 128
_KD = 10
_CHUNK = _IW * _KD


def _build():
    info = plsc.get_sparse_core_info()
    nc = info.num_cores
    nw = nc * info.num_subcores
    n_per_w = _N // nw
    nblk = n_per_w // _CHUNK
    rows_per_w = n_per_w // _IW
    mesh = plsc.VectorSubcoreMesh(core_axis_name="c", subcore_axis_name="s")

    @functools.partial(
        pl.kernel,
        mesh=mesh,
        out_type=jax.ShapeDtypeStruct((_N, _D), jnp.float32),
        compiler_params=pltpu.CompilerParams(use_tc_tiling_on_sc=False),
        scratch_types=[
            pltpu.VMEM((rows_per_w, _IW), jnp.int32),
            pltpu.VMEM((_CHUNK, _D), jnp.float32),
            pltpu.VMEM((_CHUNK, _D), jnp.float32),
            pltpu.SemaphoreType.DMA,
            pltpu.SemaphoreType.DMA,
        ],
    )
    def gather(idx_hbm, table_hbm, out_hbm, idx_v, rows0, rows1, sem0, sem1):
        wid = lax.axis_index("s") * nc + lax.axis_index("c")
        row0 = wid * rows_per_w
        base0 = wid * n_per_w

        pltpu.sync_copy(idx_hbm.at[pl.ds(row0, rows_per_w)], idx_v)

        def fire(blk, rows_v, sem):
            for j in range(_KD):
                pltpu.async_copy(
                    table_hbm.at[idx_v.at[blk * _KD + j]],
                    rows_v.at[pl.ds(j * _IW, _IW)],
                    sem,
                )

        def drain(rows_v, sem):
            pltpu.make_async_copy(out_hbm.at[pl.ds(0, _CHUNK)], rows_v, sem).wait()

        def writeback(blk, rows_v):
            pltpu.sync_copy(rows_v, out_hbm.at[pl.ds(base0 + blk * _CHUNK, _CHUNK)])

        fire(0, rows0, sem0)

        def body(g2, carry):
            g = g2 * 2
            fire(g + 1, rows1, sem1)
            drain(rows0, sem0)
            writeback(g, rows0)

            @pl.when(g + 2 < nblk)
            def _():
                fire(g + 2, rows0, sem0)

            drain(rows1, sem1)
            writeback(g + 1, rows1)
            return carry

        lax.fori_loop(0, nblk // 2, body, 0)

    return gather


_gather = _build()


def kernel(x, table):
    idx = x.reshape(_N // _IW, _IW)
    out = _gather(idx, table)
    return out.reshape(_B, _L, _D)


# speedup vs baseline: 1.5019x; 1.0000x over previous
import functools

import jax
import jax.numpy as jnp
from jax import lax
from jax.experimental import pallas as pl
from jax.experimental.pallas import tpu as pltpu
from jax.experimental.pallas import tpu_sc as plsc

_B, _L, _D = 4096, 200, 32
_N = _B * _L
_CHUNK = 1280


def _build():
    info = plsc.get_sparse_core_info()
    nc = info.num_cores
    nw = nc * info.num_subcores
    n_per_w = _N // nw
    nblk = n_per_w // _CHUNK
    mesh = plsc.VectorSubcoreMesh(core_axis_name="c", subcore_axis_name="s")

    @functools.partial(
        pl.kernel,
        mesh=mesh,
        out_type=jax.ShapeDtypeStruct((_N, _D), jnp.float32),
        compiler_params=pltpu.CompilerParams(use_tc_tiling_on_sc=False),
        scratch_types=[
            pltpu.VMEM((n_per_w,), jnp.int32),
            pltpu.VMEM((_CHUNK, _D), jnp.float32),
            pltpu.VMEM((_CHUNK, _D), jnp.float32),
            pltpu.SemaphoreType.DMA,
            pltpu.SemaphoreType.DMA,
        ],
    )
    def gather(idx_hbm, table_hbm, out_hbm, idx_v, rows0, rows1, sem0, sem1):
        wid = lax.axis_index("s") * nc + lax.axis_index("c")
        base0 = wid * n_per_w

        pltpu.sync_copy(idx_hbm.at[pl.ds(base0, n_per_w)], idx_v)

        def fire(blk, rows_v, sem):
            pltpu.async_copy(
                table_hbm.at[idx_v.at[pl.ds(blk * _CHUNK, _CHUNK)]],
                rows_v,
                sem,
            )

        def drain(rows_v, sem):
            pltpu.make_async_copy(out_hbm.at[pl.ds(0, _CHUNK)], rows_v, sem).wait()

        def writeback(blk, rows_v):
            pltpu.sync_copy(rows_v, out_hbm.at[pl.ds(base0 + blk * _CHUNK, _CHUNK)])

        fire(0, rows0, sem0)

        def body(g2, carry):
            g = g2 * 2
            fire(g + 1, rows1, sem1)
            drain(rows0, sem0)
            writeback(g, rows0)

            @pl.when(g + 2 < nblk)
            def _():
                fire(g + 2, rows0, sem0)

            drain(rows1, sem1)
            writeback(g + 1, rows1)
            return carry

        lax.fori_loop(0, nblk // 2, body, 0)

    return gather


_gather = _build()


def kernel(x, table):
    idx = x.reshape(_N)
    out = _gather(idx, table)
    return out.reshape(_B, _L, _D)
